# baseline (speedup 1.0000x reference)
import functools

import jax
import jax.numpy as jnp
from jax.experimental import pallas as pl

_B = 8


def _split_kernel(x_ref, pd_ref, ed_ref):
    x = x_ref[...]
    lane8 = jax.lax.broadcasted_iota(jnp.int32, pd_ref.shape, 2)
    pd_ref[...] = jnp.where(lane8 == 7, jnp.nan, x)
    lane9 = jax.lax.broadcasted_iota(jnp.int32, ed_ref.shape, 2)
    ed_ref[...] = jnp.where(lane9 == 8, x[:, :, 7:8], jnp.nan)


@jax.jit
def kernel(inputs):
    b, d, c = inputs.shape
    grid = (b // _B,)
    pd, ed = pl.pallas_call(
        _split_kernel,
        grid=grid,
        in_specs=[pl.BlockSpec((_B, d, 8), lambda i: (i, 0, 0))],
        out_specs=[
            pl.BlockSpec((_B, d, 8), lambda i: (i, 0, 0)),
            pl.BlockSpec((_B, d, 9), lambda i: (i, 0, 0)),
        ],
        out_shape=[
            jax.ShapeDtypeStruct((b, d, 8), jnp.float32),
            jax.ShapeDtypeStruct((b, d, 9), jnp.float32),
        ],
    )(inputs)
    return (pd, ed)


# speedup vs baseline: 4.3225x; 4.3225x over previous
import jax
import jax.numpy as jnp
from jax.experimental import pallas as pl

_B = 64


def _split_body(x_ref, pd_ref, c_ref):
    x = x_ref[...]
    bb = x.shape[0]
    l = jax.lax.broadcasted_iota(jnp.int32, x.shape, 2)
    pd_ref[...] = jnp.where(l % 8 == 7, jnp.nan, x)
    u16 = jax.lax.broadcasted_iota(jnp.int32, (bb, 64, 16), 2)
    c_ref[...] = jnp.take_along_axis(x, 8 * u16 + 7, axis=2)


def _expand_body(c_ref, ed_ref):
    x7r = c_ref[...]
    bb = x7r.shape[0]
    x7rep = jnp.repeat(x7r, 9, axis=1)
    v = jax.lax.broadcasted_iota(jnp.int32, (bb, 72, 128), 1)
    l2 = jax.lax.broadcasted_iota(jnp.int32, (bb, 72, 128), 2)
    k = 128 * v + l2
    dloc = (k // 9) % 128
    g3 = jnp.take_along_axis(x7rep, dloc, axis=2)
    ed_ref[...] = jnp.where(k % 9 == 8, g3, jnp.nan)


@jax.jit
def kernel(inputs):
    b, d, c = inputs.shape
    x3 = inputs.reshape(b, 64, 128)
    pd, c7 = pl.pallas_call(
        _split_body,
        grid=(b // _B,),
        in_specs=[pl.BlockSpec((_B, 64, 128), lambda i: (i, 0, 0))],
        out_specs=[
            pl.BlockSpec((_B, 64, 128), lambda i: (i, 0, 0)),
            pl.BlockSpec((_B, 64, 16), lambda i: (i, 0, 0)),
        ],
        out_shape=[
            jax.ShapeDtypeStruct((b, 64, 128), jnp.float32),
            jax.ShapeDtypeStruct((b, 64, 16), jnp.float32),
        ],
    )(x3)
    c7v = c7.reshape(b, 8, 128)
    ed = pl.pallas_call(
        _expand_body,
        grid=(b // _B,),
        in_specs=[pl.BlockSpec((_B, 8, 128), lambda i: (i, 0, 0))],
        out_specs=pl.BlockSpec((_B, 72, 128), lambda i: (i, 0, 0)),
        out_shape=jax.ShapeDtypeStruct((b, 72, 128), jnp.float32),
    )(c7v)
    return pd.reshape(b, d, 8), ed.reshape(b, d, 9)


# speedup vs baseline: 21.4352x; 4.9589x over previous
import jax
import jax.numpy as jnp
from jax.experimental import pallas as pl

_B = 128


def _split_body(x_ref, pd_ref, ed_ref):
    c = pl.program_id(1)

    @pl.when(c == 0)
    def _():
        x = x_ref[...]
        s = jax.lax.broadcasted_iota(jnp.int32, x.shape, 1)
        pd_ref[...] = jnp.where(s == 7, jnp.nan, x)

    @pl.when(c < 8)
    def _():
        ed_ref[...] = jnp.full(ed_ref.shape, jnp.nan, jnp.float32)

    @pl.when(c == 8)
    def _():
        ed_ref[0] = x_ref[:, 7, :]


@jax.jit
def kernel(inputs):
    b, d, ch = inputs.shape
    x_t = inputs.transpose(0, 2, 1)
    pd_t, ed_t = pl.pallas_call(
        _split_body,
        grid=(b // _B, 9),
        in_specs=[pl.BlockSpec((_B, 8, 1024), lambda i, c: (i, 0, 0))],
        out_specs=[
            pl.BlockSpec((_B, 8, 1024), lambda i, c: (i, 0, 0)),
            pl.BlockSpec((1, _B, 1024), lambda i, c: (c, i, 0)),
        ],
        out_shape=[
            jax.ShapeDtypeStruct((b, 8, 1024), jnp.float32),
            jax.ShapeDtypeStruct((9, b, 1024), jnp.float32),
        ],
    )(x_t)
    return pd_t.transpose(0, 2, 1), ed_t.transpose(1, 2, 0)


# speedup vs baseline: 28.4073x; 1.3253x over previous
import jax
import jax.numpy as jnp
from jax.experimental import pallas as pl

_B = 256


def _split_body(x_ref, pd_ref, ed_ref):
    c = pl.program_id(1)

    @pl.when(c == 0)
    def _():
        x = x_ref[...]
        s = jax.lax.broadcasted_iota(jnp.int32, x.shape, 1)
        pd_ref[...] = jnp.where(s == 7, jnp.nan, x)

    @pl.when(c < 8)
    def _():
        ed_ref[...] = jnp.full(ed_ref.shape, jnp.nan, jnp.float32)

    @pl.when(c == 8)
    def _():
        ed_ref[0] = x_ref[:, 7, :]


@jax.jit
def kernel(inputs):
    b, d, ch = inputs.shape
    x_t = inputs.transpose(0, 2, 1)
    pd_t, ed_t = pl.pallas_call(
        _split_body,
        grid=(b // _B, 9),
        in_specs=[pl.BlockSpec((_B, 8, 1024), lambda i, c: (i, 0, 0))],
        out_specs=[
            pl.BlockSpec((_B, 8, 1024), lambda i, c: (i, 0, 0)),
            pl.BlockSpec((1, _B, 1024), lambda i, c: (c, i, 0)),
        ],
        out_shape=[
            jax.ShapeDtypeStruct((b, 8, 1024), jnp.float32),
            jax.ShapeDtypeStruct((9, b, 1024), jnp.float32),
        ],
    )(x_t)
    return pd_t.transpose(0, 2, 1), ed_t.transpose(1, 2, 0)


# speedup vs baseline: 35.8611x; 1.2624x over previous
import jax
import jax.numpy as jnp
from jax.experimental import pallas as pl
from jax.experimental.pallas import tpu as pltpu

_B = 512


def _split_body(x_ref, pd_ref, ed_ref, c7_ref):
    c = pl.program_id(1)
    b8 = x_ref.shape[0]

    @pl.when(c < 8)
    def _():
        x = x_ref[...]
        s = jax.lax.broadcasted_iota(jnp.int32, x.shape, 1)
        pd_ref[...] = jnp.where(s == 7, jnp.nan, x)
        c7_ref[pl.ds(c * b8, b8), :] = x[:, 7, :]
        ed_ref[...] = jnp.full(ed_ref.shape, jnp.nan, jnp.float32)

    @pl.when(c == 8)
    def _():
        ed_ref[0] = c7_ref[...]


@jax.jit
def kernel(inputs):
    b, d, ch = inputs.shape
    b8 = _B // 8
    x_t = inputs.transpose(0, 2, 1)
    pd_t, ed_t = pl.pallas_call(
        _split_body,
        grid=(b // _B, 9),
        in_specs=[
            pl.BlockSpec((b8, 8, 1024), lambda i, c: (8 * i + jnp.minimum(c, 7), 0, 0))
        ],
        out_specs=[
            pl.BlockSpec((b8, 8, 1024), lambda i, c: (8 * i + jnp.minimum(c, 7), 0, 0)),
            pl.BlockSpec((1, _B, 1024), lambda i, c: (c, i, 0)),
        ],
        out_shape=[
            jax.ShapeDtypeStruct((b, 8, 1024), jnp.float32),
            jax.ShapeDtypeStruct((9, b, 1024), jnp.float32),
        ],
        scratch_shapes=[pltpu.VMEM((_B, 1024), jnp.float32)],
    )(x_t)
    return pd_t.transpose(0, 2, 1), ed_t.transpose(1, 2, 0)


# speedup vs baseline: 38.8136x; 1.0823x over previous
import jax
import jax.numpy as jnp
from jax.experimental import pallas as pl
from jax.experimental.pallas import tpu as pltpu

_B = 1024


def _split_body(x_ref, pd_ref, ed_ref, c7_ref):
    c = pl.program_id(1)
    b8 = x_ref.shape[0]

    @pl.when(c < 8)
    def _():
        x = x_ref[...]
        s = jax.lax.broadcasted_iota(jnp.int32, x.shape, 1)
        pd_ref[...] = jnp.where(s == 7, jnp.nan, x)
        c7_ref[pl.ds(c * b8, b8), :] = x[:, 7, :]
        ed_ref[...] = jnp.full(ed_ref.shape, jnp.nan, jnp.float32)

    @pl.when(c == 8)
    def _():
        ed_ref[0] = c7_ref[...]


@jax.jit
def kernel(inputs):
    b, d, ch = inputs.shape
    b8 = _B // 8
    x_t = inputs.transpose(0, 2, 1)
    pd_t, ed_t = pl.pallas_call(
        _split_body,
        grid=(b // _B, 9),
        in_specs=[
            pl.BlockSpec((b8, 8, 1024), lambda i, c: (8 * i + jnp.minimum(c, 7), 0, 0))
        ],
        out_specs=[
            pl.BlockSpec((b8, 8, 1024), lambda i, c: (8 * i + jnp.minimum(c, 7), 0, 0)),
            pl.BlockSpec((1, _B, 1024), lambda i, c: (c, i, 0)),
        ],
        out_shape=[
            jax.ShapeDtypeStruct((b, 8, 1024), jnp.float32),
            jax.ShapeDtypeStruct((9, b, 1024), jnp.float32),
        ],
        scratch_shapes=[pltpu.VMEM((_B, 1024), jnp.float32)],
    )(x_t)
    return pd_t.transpose(0, 2, 1), ed_t.transpose(1, 2, 0)


# speedup vs baseline: 38.8274x; 1.0004x over previous
import jax
import jax.numpy as jnp
from jax.experimental import pallas as pl
from jax.experimental.pallas import tpu as pltpu

_B = 2048


def _split_body(x_ref, pd_ref, ed_ref, c7_ref):
    c = pl.program_id(1)
    b16 = x_ref.shape[0]
    half = ed_ref.shape[1]

    @pl.when(c < 16)
    def _():
        x = x_ref[...]
        s = jax.lax.broadcasted_iota(jnp.int32, x.shape, 1)
        pd_ref[...] = jnp.where(s == 7, jnp.nan, x)
        c7_ref[pl.ds(c * b16, b16), :] = x[:, 7, :]
        ed_ref[...] = jnp.full(ed_ref.shape, jnp.nan, jnp.float32)

    @pl.when(c >= 16)
    def _():
        h = c - 16
        ed_ref[0] = c7_ref[pl.ds(h * half, half), :]


@jax.jit
def kernel(inputs):
    b, d, ch = inputs.shape
    b16 = _B // 16
    half = _B // 2
    x_t = inputs.transpose(0, 2, 1)
    pd_t, ed_t = pl.pallas_call(
        _split_body,
        grid=(b // _B, 18),
        in_specs=[
            pl.BlockSpec(
                (b16, 8, 1024), lambda i, c: (16 * i + jnp.minimum(c, 15), 0, 0)
            )
        ],
        out_specs=[
            pl.BlockSpec(
                (b16, 8, 1024), lambda i, c: (16 * i + jnp.minimum(c, 15), 0, 0)
            ),
            pl.BlockSpec((1, half, 1024), lambda i, c: (c // 2, 2 * i + c % 2, 0)),
        ],
        out_shape=[
            jax.ShapeDtypeStruct((b, 8, 1024), jnp.float32),
            jax.ShapeDtypeStruct((9, b, 1024), jnp.float32),
        ],
        scratch_shapes=[pltpu.VMEM((_B, 1024), jnp.float32)],
    )(x_t)
    return pd_t.transpose(0, 2, 1), ed_t.transpose(1, 2, 0)


# speedup vs baseline: 38.8484x; 1.0005x over previous
import jax
import jax.numpy as jnp
from jax.experimental import pallas as pl
from jax.experimental.pallas import tpu as pltpu

_B = 1024


def _split_body(x_ref, pd_ref, ed_ref, c7_ref):
    c = pl.program_id(1)
    b8 = x_ref.shape[0]

    @pl.when(c < 8)
    def _():
        x = x_ref[...]
        s = jax.lax.broadcasted_iota(jnp.int32, x.shape, 1)
        pd_ref[...] = jnp.where(s == 7, jnp.nan, x)
        c7_ref[pl.ds(c * b8, b8), :] = x[:, 7, :]
        ed_ref[...] = jnp.full(ed_ref.shape, jnp.nan, jnp.float32)

    @pl.when(c == 8)
    def _():
        ed_ref[0] = c7_ref[...]


@jax.jit
def kernel(inputs):
    b, d, ch = inputs.shape
    b8 = _B // 8
    x_t = inputs.transpose(0, 2, 1)
    pd_t, ed_t = pl.pallas_call(
        _split_body,
        grid=(b // _B, 9),
        in_specs=[
            pl.BlockSpec((b8, 8, 1024), lambda i, c: (8 * i + jnp.minimum(c, 7), 0, 0))
        ],
        out_specs=[
            pl.BlockSpec((b8, 8, 1024), lambda i, c: (8 * i + jnp.minimum(c, 7), 0, 0)),
            pl.BlockSpec((1, _B, 1024), lambda i, c: (c, i, 0)),
        ],
        out_shape=[
            jax.ShapeDtypeStruct((b, 8, 1024), jnp.float32),
            jax.ShapeDtypeStruct((9, b, 1024), jnp.float32),
        ],
        scratch_shapes=[pltpu.VMEM((_B, 1024), jnp.float32)],
    )(x_t)
    return pd_t.transpose(0, 2, 1), ed_t.transpose(1, 2, 0)
